# baseline (speedup 1.0000x reference)
import functools

import jax
import jax.numpy as jnp
from jax import lax
from jax.experimental import pallas as pl
from jax.experimental.pallas import tpu as pltpu
from jax.experimental.pallas import tpu_sc as plsc

A_N = 20000
L_N = 80000
N_NODES = A_N + L_N
H = 64
T_STEPS = 10
E_AA = 400000
E_AL = 300000
E_REAL = E_AA + 2 * E_AL
E_PAD = 1 << 20
N_PAD = 102400
NC = 2
NS = 16
ROWS_PER_TILE = N_PAD // NS
EDGE_K = 512
ZB = 256
DEG_K = 2048
LSTM_BLOCK = 2000
R_G = 800


def _lstm_body(x_ref, wbig_ref, whhT_ref, b_ref, out_ref, *, mean_pool):
    xp = jnp.dot(x_ref[...], wbig_ref[...], preferred_element_type=jnp.float32)
    b = b_ref[...]
    whhT = whhT_ref[...]
    r = x_ref.shape[0]
    h = jnp.zeros((r, H), jnp.float32)
    c = jnp.zeros((r, H), jnp.float32)
    acc = jnp.zeros((r, H), jnp.float32)
    for t in range(T_STEPS):
        g = xp[:, t * 4 * H:(t + 1) * 4 * H]
        g = g + jnp.dot(h, whhT, preferred_element_type=jnp.float32) + b
        i = jax.nn.sigmoid(g[:, :H])
        f = jax.nn.sigmoid(g[:, H:2 * H])
        gg = jnp.tanh(g[:, 2 * H:3 * H])
        o = jax.nn.sigmoid(g[:, 3 * H:])
        c = f * c + i * gg
        h = o * jnp.tanh(c)
        if mean_pool:
            acc = acc + h
    out_ref[...] = acc * (1.0 / T_STEPS) if mean_pool else h


def _run_lstm(x, Wih, Whh, bih, bhh, mean_pool, block_rows):
    b_sz, t_sz, d = x.shape
    x2 = x.reshape(b_sz, t_sz * d)
    wihT = Wih.T
    eye = jnp.eye(t_sz, dtype=jnp.float32)
    wbig = (eye[:, None, :, None] * wihT[None, :, None, :]).reshape(
        t_sz * d, t_sz * 4 * H)
    b = (bih + bhh).reshape(1, 4 * H)
    grid = (b_sz // block_rows,)
    return pl.pallas_call(
        functools.partial(_lstm_body, mean_pool=mean_pool),
        grid=grid,
        in_specs=[
            pl.BlockSpec((block_rows, t_sz * d), lambda i: (i, 0)),
            pl.BlockSpec((t_sz * d, t_sz * 4 * H), lambda i: (0, 0)),
            pl.BlockSpec((H, 4 * H), lambda i: (0, 0)),
            pl.BlockSpec((1, 4 * H), lambda i: (0, 0)),
        ],
        out_specs=pl.BlockSpec((block_rows, H), lambda i: (i, 0)),
        out_shape=jax.ShapeDtypeStruct((b_sz, H), jnp.float32),
    )(x2, wbig, Whh.T, b)


def _ew_body(l_ref, w_ref, b_ref, o_ref):
    o_ref[...] = jax.nn.sigmoid(l_ref[...] * w_ref[0, 0] + b_ref[0, 0])


def _edge_weights(lens, we, be):
    n = lens.shape[0]
    lp = ((n + 127) // 128 + 7) // 8 * 8 * 128
    l2 = jnp.pad(lens, (0, lp - n)).reshape(lp // 128, 128)
    w2 = pl.pallas_call(
        _ew_body,
        grid=(1,),
        in_specs=[
            pl.BlockSpec((lp // 128, 128), lambda i: (0, 0)),
            pl.BlockSpec((1, 1), lambda i: (0, 0)),
            pl.BlockSpec((1, 1), lambda i: (0, 0)),
        ],
        out_specs=pl.BlockSpec((lp // 128, 128), lambda i: (0, 0)),
        out_shape=jax.ShapeDtypeStruct((lp // 128, 128), jnp.float32),
    )(l2, we.reshape(1, 1), be.reshape(1, 1))
    return w2.reshape(-1)[:n]


def _glue1_body(a_ref, l_ref, w_ref, d0_ref, d1_ref, y_ref, xw_ref, dinv_ref,
                *, a_blocks):
    x = jnp.where(pl.program_id(0) < a_blocks, a_ref[...], l_ref[...])
    deg = d0_ref[...] + d1_ref[...] + 1.0
    dinv = lax.rsqrt(deg)
    xw = jnp.dot(x, w_ref[...], preferred_element_type=jnp.float32)
    xw_ref[...] = xw
    dinv_ref[...] = dinv
    y_ref[...] = xw * dinv


def _glue1(agent_emb, lane_emb, w1T, d0, d1):
    r = R_G
    a_blocks = A_N // r
    l_blocks = L_N // r
    return pl.pallas_call(
        functools.partial(_glue1_body, a_blocks=a_blocks),
        grid=(N_PAD // r,),
        in_specs=[
            pl.BlockSpec((r, H), lambda i: (jnp.minimum(i, a_blocks - 1), 0)),
            pl.BlockSpec((r, H),
                         lambda i: (jnp.clip(i - a_blocks, 0, l_blocks - 1),
                                    0)),
            pl.BlockSpec((H, H), lambda i: (0, 0)),
            pl.BlockSpec((r, 1), lambda i: (i, 0)),
            pl.BlockSpec((r, 1), lambda i: (i, 0)),
        ],
        out_specs=[
            pl.BlockSpec((r, H), lambda i: (i, 0)),
            pl.BlockSpec((r, H), lambda i: (i, 0)),
            pl.BlockSpec((r, 1), lambda i: (i, 0)),
        ],
        out_shape=[
            jax.ShapeDtypeStruct((N_PAD, H), jnp.float32),
            jax.ShapeDtypeStruct((N_PAD, H), jnp.float32),
            jax.ShapeDtypeStruct((N_PAD, 1), jnp.float32),
        ],
    )(agent_emb, lane_emb, w1T, d0, d1)


def _glue2_body(s_ref, xw1_ref, dinv_ref, w2T_ref, b1_ref, bns_ref, bnb_ref,
                y2_ref, xw2_ref):
    dinv = dinv_ref[...]
    o1 = dinv * s_ref[...] + (dinv * dinv) * xw1_ref[...] + b1_ref[...]
    hh = jnp.maximum(o1 * bns_ref[...] + bnb_ref[...], 0.0)
    xw2 = jnp.dot(hh, w2T_ref[...], preferred_element_type=jnp.float32)
    xw2_ref[...] = xw2
    y2_ref[...] = xw2 * dinv


def _glue2(s1, xw1, dinv, w2T, b1, bns, bnb):
    r = R_G
    return pl.pallas_call(
        _glue2_body,
        grid=(N_PAD // r,),
        in_specs=[
            pl.BlockSpec((r, H), lambda i: (i, 0)),
            pl.BlockSpec((r, H), lambda i: (i, 0)),
            pl.BlockSpec((r, 1), lambda i: (i, 0)),
            pl.BlockSpec((H, H), lambda i: (0, 0)),
            pl.BlockSpec((1, H), lambda i: (0, 0)),
            pl.BlockSpec((1, H), lambda i: (0, 0)),
            pl.BlockSpec((1, H), lambda i: (0, 0)),
        ],
        out_specs=[
            pl.BlockSpec((r, H), lambda i: (i, 0)),
            pl.BlockSpec((r, H), lambda i: (i, 0)),
        ],
        out_shape=[
            jax.ShapeDtypeStruct((N_PAD, H), jnp.float32),
            jax.ShapeDtypeStruct((N_PAD, H), jnp.float32),
        ],
    )(s1, xw1, dinv, w2T, b1, bns, bnb)


def _glue3_body(s_ref, xw2_ref, dinv_ref, b2_ref, out_ref):
    dinv = dinv_ref[...]
    out_ref[...] = dinv * s_ref[...] + (dinv * dinv) * xw2_ref[...] + b2_ref[...]


def _glue3(s2, xw2, dinv, b2):
    r = R_G
    return pl.pallas_call(
        _glue3_body,
        grid=(N_PAD // r,),
        in_specs=[
            pl.BlockSpec((r, H), lambda i: (i, 0)),
            pl.BlockSpec((r, H), lambda i: (i, 0)),
            pl.BlockSpec((r, 1), lambda i: (i, 0)),
            pl.BlockSpec((1, H), lambda i: (0, 0)),
        ],
        out_specs=pl.BlockSpec((r, H), lambda i: (i, 0)),
        out_shape=jax.ShapeDtypeStruct((N_PAD, H), jnp.float32),
    )(s2, xw2, dinv, b2)


def _sc_degree_body(dst_hbm, ew_hbm, out_hbm, idx_v, val_v, zb_v, acc_sh):
    cid = lax.axis_index("c")
    sid = lax.axis_index("s")

    @pl.loop(0, ROWS_PER_TILE // 16)
    def _(i):
        zb_v[pl.ds(i * 16, 16)] = jnp.zeros((16,), jnp.float32)

    my0 = pl.multiple_of(sid * ROWS_PER_TILE, 8)
    pltpu.sync_copy(zb_v, acc_sh.at[pl.ds(my0, ROWS_PER_TILE)])
    plsc.subcore_barrier()

    epw = E_PAD // (NC * NS)
    wid = cid * NS + sid

    @pl.loop(0, epw // DEG_K)
    def _(k):
        b = pl.multiple_of(wid * epw + k * DEG_K, 8)
        pltpu.sync_copy(dst_hbm.at[pl.ds(b, DEG_K)], idx_v)
        pltpu.sync_copy(ew_hbm.at[pl.ds(b, DEG_K)], val_v)
        pltpu.sync_copy(val_v, acc_sh.at[idx_v], add=True)

    plsc.subcore_barrier()
    pltpu.sync_copy(acc_sh.at[pl.ds(my0, ROWS_PER_TILE)],
                    out_hbm.at[cid, pl.ds(my0, ROWS_PER_TILE)])


def _sc_degree(dst, ew):
    mesh = plsc.VectorSubcoreMesh(core_axis_name="c", subcore_axis_name="s")
    return pl.kernel(
        _sc_degree_body,
        out_type=jax.ShapeDtypeStruct((NC, N_PAD), jnp.float32),
        mesh=mesh,
        scratch_types=[
            pltpu.VMEM((DEG_K,), jnp.int32),
            pltpu.VMEM((DEG_K,), jnp.float32),
            pltpu.VMEM((ROWS_PER_TILE,), jnp.float32),
            pltpu.VMEM_SHARED((N_PAD,), jnp.float32),
        ],
    )(dst, ew)


def _sc_conv_body(y_hbm, src_hbm, dst_hbm, ew_hbm, out_hbm,
                  src0, src1, dst0, dst1, ew0, ew1, idx0, idx1,
                  sidx0, sidx1, rows0, rows1, zb_v, acc_sh,
                  msem0, msem1, gsem0, gsem1, ssem0, ssem1):
    cid = lax.axis_index("c")
    sid = lax.axis_index("s")
    srcb = (src0, src1)
    dstb = (dst0, dst1)
    ewb = (ew0, ew1)
    idxb = (idx0, idx1)
    sidxb = (sidx0, sidx1)
    rowsb = (rows0, rows1)
    msem = (msem0, msem1)
    gsem = (gsem0, gsem1)
    ssem = (ssem0, ssem1)

    @pl.loop(0, ZB)
    def _(i):
        zb_v[i, :] = jnp.zeros((16,), jnp.float32)

    epw = E_PAD // NS
    n2 = epw // EDGE_K // 2
    tile0 = pl.multiple_of(sid * epw, 8)

    def load_meta(b, k):
        base = pl.multiple_of(tile0 + k * EDGE_K, 8)
        pltpu.async_copy(src_hbm.at[pl.ds(base, EDGE_K)], srcb[b], msem[b])
        pltpu.async_copy(dst_hbm.at[pl.ds(base, EDGE_K)], dstb[b], msem[b])
        pltpu.async_copy(ew_hbm.at[pl.ds(base, EDGE_K)], ewb[b], msem[b])

    def wait_meta(b):
        pltpu.make_async_copy(src_hbm.at[pl.ds(0, EDGE_K)], srcb[b],
                              msem[b]).wait()
        pltpu.make_async_copy(dst_hbm.at[pl.ds(0, EDGE_K)], dstb[b],
                              msem[b]).wait()
        pltpu.make_async_copy(ew_hbm.at[pl.ds(0, EDGE_K)], ewb[b],
                              msem[b]).wait()

    def fire_gather(b):
        pltpu.async_copy(y_hbm.at[idxb[b]], rowsb[b], gsem[b])

    def wait_gather(b):
        pltpu.make_async_copy(y_hbm.at[idxb[b]], rowsb[b], gsem[b]).wait()

    def fire_scatter(b):
        pltpu.async_copy(rowsb[b], acc_sh.at[sidxb[b]], ssem[b], add=True)

    def wait_scatter(b):
        pltpu.make_async_copy(rowsb[b], acc_sh.at[sidxb[b]], ssem[b]).wait()

    def compute_idx(b, c):
        @plsc.parallel_loop(0, EDGE_K // 16, unroll=4)
        def _(i):
            o = pl.multiple_of(i * 16, 8)
            idxb[b][pl.ds(o, 16)] = srcb[b][pl.ds(o, 16)] * 4 + c

    def scale_and_scatter(b):
        @plsc.parallel_loop(0, EDGE_K // 16, unroll=2)
        def _(i):
            o = pl.multiple_of(i * 16, 8)
            w = ewb[b][pl.ds(o, 16)]
            sidxb[b][pl.ds(o, 16)] = dstb[b][pl.ds(o, 16)]
            for j in range(16):
                rowsb[b][o + j, :] = rowsb[b][o + j, :] * w[j]
        fire_scatter(b)

    for kp in range(2):
        c = cid * 2 + kp

        @pl.loop(0, ROWS_PER_TILE // ZB)
        def _(j):
            r0 = pl.multiple_of(sid * ROWS_PER_TILE + j * ZB, 8)
            pltpu.async_copy(zb_v, acc_sh.at[pl.ds(r0, ZB)], msem0)

        @pl.loop(0, ROWS_PER_TILE // ZB)
        def _(j):
            pltpu.make_async_copy(zb_v, acc_sh.at[pl.ds(0, ZB)],
                                  msem0).wait()

        plsc.subcore_barrier()

        load_meta(0, 0)
        load_meta(1, 1)
        wait_meta(0)
        compute_idx(0, c)
        fire_gather(0)

        @pl.loop(0, n2)
        def _(j):
            wait_meta(1)
            compute_idx(1, c)

            @pl.when(j > 0)
            def _():
                wait_scatter(1)

            fire_gather(1)
            wait_gather(0)
            scale_and_scatter(0)

            @pl.when(j < n2 - 1)
            def _():
                load_meta(0, 2 * j + 2)
                wait_meta(0)
                compute_idx(0, c)
                wait_scatter(0)
                fire_gather(0)

            wait_gather(1)
            scale_and_scatter(1)

            @pl.when(j < n2 - 1)
            def _():
                load_meta(1, 2 * j + 3)

        wait_scatter(0)
        wait_scatter(1)
        plsc.subcore_barrier()

        @pl.loop(0, ROWS_PER_TILE // ZB)
        def _(j):
            r0 = pl.multiple_of(sid * ROWS_PER_TILE + j * ZB, 8)
            pltpu.async_copy(acc_sh.at[pl.ds(r0, ZB)],
                             out_hbm.at[pl.ds(r0, ZB), pl.ds(c * 16, 16)],
                             msem0)

        @pl.loop(0, ROWS_PER_TILE // ZB)
        def _(j):
            pltpu.make_async_copy(acc_sh.at[pl.ds(0, ZB)],
                                  out_hbm.at[pl.ds(0, ZB), pl.ds(0, 16)],
                                  msem0).wait()


def _sc_conv(y_flat, src, dst, ew):
    mesh = plsc.VectorSubcoreMesh(core_axis_name="c", subcore_axis_name="s")
    return pl.kernel(
        _sc_conv_body,
        out_type=jax.ShapeDtypeStruct((N_PAD, H), jnp.float32),
        mesh=mesh,
        compiler_params=pltpu.CompilerParams(use_tc_tiling_on_sc=False),
        scratch_types=[
            pltpu.VMEM((EDGE_K,), jnp.int32),
            pltpu.VMEM((EDGE_K,), jnp.int32),
            pltpu.VMEM((EDGE_K,), jnp.int32),
            pltpu.VMEM((EDGE_K,), jnp.int32),
            pltpu.VMEM((EDGE_K,), jnp.float32),
            pltpu.VMEM((EDGE_K,), jnp.float32),
            pltpu.VMEM((EDGE_K,), jnp.int32),
            pltpu.VMEM((EDGE_K,), jnp.int32),
            pltpu.VMEM((EDGE_K,), jnp.int32),
            pltpu.VMEM((EDGE_K,), jnp.int32),
            pltpu.VMEM((EDGE_K, 16), jnp.float32),
            pltpu.VMEM((EDGE_K, 16), jnp.float32),
            pltpu.VMEM((ZB, 16), jnp.float32),
            pltpu.VMEM_SHARED((N_PAD, 16), jnp.float32),
            pltpu.SemaphoreType.DMA,
            pltpu.SemaphoreType.DMA,
            pltpu.SemaphoreType.DMA,
            pltpu.SemaphoreType.DMA,
            pltpu.SemaphoreType.DMA,
            pltpu.SemaphoreType.DMA,
        ],
    )(y_flat, src, dst, ew)


def kernel(agent_hist, lane_nodes, edge_index_aa, edges_length, edge_index_al,
           edges_length_al, Wih_a, Whh_a, bih_a, bhh_a, Wih_l, Whh_l, bih_l,
           bhh_l, we, be, W1, b1, W2, b2, bn_gamma, bn_beta, bn_mean, bn_var):
    agent_emb = _run_lstm(agent_hist, Wih_a, Whh_a, bih_a, bhh_a, False, LSTM_BLOCK)
    lane_emb = _run_lstm(lane_nodes, Wih_l, Whh_l, bih_l, bhh_l, True, LSTM_BLOCK)

    lens = jnp.concatenate([edges_length, edges_length_al])
    w_all = _edge_weights(lens, we, be)
    w_aa = w_all[:E_AA]
    w_al = w_all[E_AA:]

    al_a = edge_index_al[0].astype(jnp.int32)
    al_l = edge_index_al[1].astype(jnp.int32) + A_N
    n_pad_e = E_PAD - E_REAL
    pad_idx = jnp.arange(n_pad_e, dtype=jnp.int32) % N_NODES
    src = jnp.concatenate([edge_index_aa[0].astype(jnp.int32), al_a, al_l,
                           pad_idx])
    dst = jnp.concatenate([edge_index_aa[1].astype(jnp.int32), al_l, al_a,
                           pad_idx])
    ew = jnp.concatenate([w_aa, w_al, w_al,
                          jnp.zeros((n_pad_e,), jnp.float32)])

    degp = _sc_degree(dst, ew)
    d0 = degp[0].reshape(N_PAD, 1)
    d1 = degp[1].reshape(N_PAD, 1)

    y1, xw1, dinv = _glue1(agent_emb, lane_emb, W1.T, d0, d1)
    s1 = _sc_conv(y1.reshape(4 * N_PAD, 16), src, dst, ew)

    bn_scale = bn_gamma / jnp.sqrt(bn_var + 1e-5)
    bn_shift = bn_beta - bn_mean * bn_scale
    y2, xw2 = _glue2(s1, xw1, dinv, W2.T, b1.reshape(1, H),
                     bn_scale.reshape(1, H), bn_shift.reshape(1, H))
    s2 = _sc_conv(y2.reshape(4 * N_PAD, 16), src, dst, ew)
    x_out = _glue3(s2, xw2, dinv, b2.reshape(1, H))

    agent_map = x_out[:A_N]
    lane_out = x_out[A_N:N_NODES]
    return (agent_emb, agent_map, lane_emb, lane_out)


# speedup vs baseline: 1.0123x; 1.0123x over previous
import functools

import jax
import jax.numpy as jnp
from jax import lax
from jax.experimental import pallas as pl
from jax.experimental.pallas import tpu as pltpu
from jax.experimental.pallas import tpu_sc as plsc

A_N = 20000
L_N = 80000
N_NODES = A_N + L_N
H = 64
T_STEPS = 10
E_AA = 400000
E_AL = 300000
E_REAL = E_AA + 2 * E_AL
E_PAD = 1 << 20
N_PAD = 102400
NC = 2
NS = 16
ROWS_PER_TILE = N_PAD // NS
EDGE_K = 512
ZB = 256
DEG_K = 2048
LSTM_BLOCK = 2000
R_G = 1024


def _lstm_body(x_ref, wbig_ref, whhT_ref, b_ref, out_ref, *, mean_pool):
    xp = jnp.dot(x_ref[...], wbig_ref[...], preferred_element_type=jnp.float32)
    b = b_ref[...]
    whhT = whhT_ref[...]
    r = x_ref.shape[0]
    h = jnp.zeros((r, H), jnp.float32)
    c = jnp.zeros((r, H), jnp.float32)
    acc = jnp.zeros((r, H), jnp.float32)
    for t in range(T_STEPS):
        g = xp[:, t * 4 * H:(t + 1) * 4 * H]
        g = g + jnp.dot(h, whhT, preferred_element_type=jnp.float32) + b
        i = jax.nn.sigmoid(g[:, :H])
        f = jax.nn.sigmoid(g[:, H:2 * H])
        gg = jnp.tanh(g[:, 2 * H:3 * H])
        o = jax.nn.sigmoid(g[:, 3 * H:])
        c = f * c + i * gg
        h = o * jnp.tanh(c)
        if mean_pool:
            acc = acc + h
    out_ref[...] = acc * (1.0 / T_STEPS) if mean_pool else h


def _run_lstm(x, Wih, Whh, bih, bhh, mean_pool, block_rows):
    b_sz, t_sz, d = x.shape
    x2 = x.reshape(b_sz, t_sz * d)
    wihT = Wih.T
    eye = jnp.eye(t_sz, dtype=jnp.float32)
    wbig = (eye[:, None, :, None] * wihT[None, :, None, :]).reshape(
        t_sz * d, t_sz * 4 * H)
    b = (bih + bhh).reshape(1, 4 * H)
    grid = (b_sz // block_rows,)
    return pl.pallas_call(
        functools.partial(_lstm_body, mean_pool=mean_pool),
        grid=grid,
        in_specs=[
            pl.BlockSpec((block_rows, t_sz * d), lambda i: (i, 0)),
            pl.BlockSpec((t_sz * d, t_sz * 4 * H), lambda i: (0, 0)),
            pl.BlockSpec((H, 4 * H), lambda i: (0, 0)),
            pl.BlockSpec((1, 4 * H), lambda i: (0, 0)),
        ],
        out_specs=pl.BlockSpec((block_rows, H), lambda i: (i, 0)),
        out_shape=jax.ShapeDtypeStruct((b_sz, H), jnp.float32),
    )(x2, wbig, Whh.T, b)


def _ew_body(l_ref, w_ref, b_ref, o_ref):
    o_ref[...] = jax.nn.sigmoid(l_ref[...] * w_ref[0, 0] + b_ref[0, 0])


def _edge_weights(lens, we, be):
    n = lens.shape[0]
    lp = ((n + 127) // 128 + 7) // 8 * 8 * 128
    l2 = jnp.pad(lens, (0, lp - n)).reshape(lp // 128, 128)
    w2 = pl.pallas_call(
        _ew_body,
        grid=(1,),
        in_specs=[
            pl.BlockSpec((lp // 128, 128), lambda i: (0, 0)),
            pl.BlockSpec((1, 1), lambda i: (0, 0)),
            pl.BlockSpec((1, 1), lambda i: (0, 0)),
        ],
        out_specs=pl.BlockSpec((lp // 128, 128), lambda i: (0, 0)),
        out_shape=jax.ShapeDtypeStruct((lp // 128, 128), jnp.float32),
    )(l2, we.reshape(1, 1), be.reshape(1, 1))
    return w2.reshape(-1)[:n]


def _glue1_body(x_ref, w_ref, d0_ref, d1_ref, y_ref, xw_ref, dinv_ref):
    deg = d0_ref[...] + d1_ref[...] + 1.0
    dinv = lax.rsqrt(deg)
    xw = jnp.dot(x_ref[...], w_ref[...], preferred_element_type=jnp.float32)
    xw_ref[...] = xw
    dinv_ref[...] = dinv
    y_ref[...] = xw * dinv


def _glue1(x_pad, w1T, d0, d1):
    r = R_G
    return pl.pallas_call(
        _glue1_body,
        grid=(N_PAD // r,),
        in_specs=[
            pl.BlockSpec((r, H), lambda i: (i, 0)),
            pl.BlockSpec((H, H), lambda i: (0, 0)),
            pl.BlockSpec((r, 1), lambda i: (i, 0)),
            pl.BlockSpec((r, 1), lambda i: (i, 0)),
        ],
        out_specs=[
            pl.BlockSpec((r, H), lambda i: (i, 0)),
            pl.BlockSpec((r, H), lambda i: (i, 0)),
            pl.BlockSpec((r, 1), lambda i: (i, 0)),
        ],
        out_shape=[
            jax.ShapeDtypeStruct((N_PAD, H), jnp.float32),
            jax.ShapeDtypeStruct((N_PAD, H), jnp.float32),
            jax.ShapeDtypeStruct((N_PAD, 1), jnp.float32),
        ],
    )(x_pad, w1T, d0, d1)


def _glue2_body(s_ref, xw1_ref, dinv_ref, w2T_ref, b1_ref, bns_ref, bnb_ref,
                y2_ref, xw2_ref):
    dinv = dinv_ref[...]
    o1 = dinv * s_ref[...] + (dinv * dinv) * xw1_ref[...] + b1_ref[...]
    hh = jnp.maximum(o1 * bns_ref[...] + bnb_ref[...], 0.0)
    xw2 = jnp.dot(hh, w2T_ref[...], preferred_element_type=jnp.float32)
    xw2_ref[...] = xw2
    y2_ref[...] = xw2 * dinv


def _glue2(s1, xw1, dinv, w2T, b1, bns, bnb):
    r = R_G
    return pl.pallas_call(
        _glue2_body,
        grid=(N_PAD // r,),
        in_specs=[
            pl.BlockSpec((r, H), lambda i: (i, 0)),
            pl.BlockSpec((r, H), lambda i: (i, 0)),
            pl.BlockSpec((r, 1), lambda i: (i, 0)),
            pl.BlockSpec((H, H), lambda i: (0, 0)),
            pl.BlockSpec((1, H), lambda i: (0, 0)),
            pl.BlockSpec((1, H), lambda i: (0, 0)),
            pl.BlockSpec((1, H), lambda i: (0, 0)),
        ],
        out_specs=[
            pl.BlockSpec((r, H), lambda i: (i, 0)),
            pl.BlockSpec((r, H), lambda i: (i, 0)),
        ],
        out_shape=[
            jax.ShapeDtypeStruct((N_PAD, H), jnp.float32),
            jax.ShapeDtypeStruct((N_PAD, H), jnp.float32),
        ],
    )(s1, xw1, dinv, w2T, b1, bns, bnb)


def _glue3_body(s_ref, xw2_ref, dinv_ref, b2_ref, out_ref):
    dinv = dinv_ref[...]
    out_ref[...] = dinv * s_ref[...] + (dinv * dinv) * xw2_ref[...] + b2_ref[...]


def _glue3(s2, xw2, dinv, b2):
    r = R_G
    return pl.pallas_call(
        _glue3_body,
        grid=(N_PAD // r,),
        in_specs=[
            pl.BlockSpec((r, H), lambda i: (i, 0)),
            pl.BlockSpec((r, H), lambda i: (i, 0)),
            pl.BlockSpec((r, 1), lambda i: (i, 0)),
            pl.BlockSpec((1, H), lambda i: (0, 0)),
        ],
        out_specs=pl.BlockSpec((r, H), lambda i: (i, 0)),
        out_shape=jax.ShapeDtypeStruct((N_PAD, H), jnp.float32),
    )(s2, xw2, dinv, b2)


def _sc_degree_body(dst_hbm, ew_hbm, out_hbm, idx_v, val_v, zb_v, acc_sh):
    cid = lax.axis_index("c")
    sid = lax.axis_index("s")

    @pl.loop(0, ROWS_PER_TILE // 16)
    def _(i):
        zb_v[pl.ds(i * 16, 16)] = jnp.zeros((16,), jnp.float32)

    my0 = pl.multiple_of(sid * ROWS_PER_TILE, 8)
    pltpu.sync_copy(zb_v, acc_sh.at[pl.ds(my0, ROWS_PER_TILE)])
    plsc.subcore_barrier()

    epw = E_PAD // (NC * NS)
    wid = cid * NS + sid

    @pl.loop(0, epw // DEG_K)
    def _(k):
        b = pl.multiple_of(wid * epw + k * DEG_K, 8)
        pltpu.sync_copy(dst_hbm.at[pl.ds(b, DEG_K)], idx_v)
        pltpu.sync_copy(ew_hbm.at[pl.ds(b, DEG_K)], val_v)
        pltpu.sync_copy(val_v, acc_sh.at[idx_v], add=True)

    plsc.subcore_barrier()
    pltpu.sync_copy(acc_sh.at[pl.ds(my0, ROWS_PER_TILE)],
                    out_hbm.at[cid, pl.ds(my0, ROWS_PER_TILE)])


def _sc_degree(dst, ew):
    mesh = plsc.VectorSubcoreMesh(core_axis_name="c", subcore_axis_name="s")
    return pl.kernel(
        _sc_degree_body,
        out_type=jax.ShapeDtypeStruct((NC, N_PAD), jnp.float32),
        mesh=mesh,
        scratch_types=[
            pltpu.VMEM((DEG_K,), jnp.int32),
            pltpu.VMEM((DEG_K,), jnp.float32),
            pltpu.VMEM((ROWS_PER_TILE,), jnp.float32),
            pltpu.VMEM_SHARED((N_PAD,), jnp.float32),
        ],
    )(dst, ew)


def _sc_conv_body(y_hbm, src_hbm, dst_hbm, ew_hbm, out_hbm,
                  src0, src1, dst0, dst1, ew0, ew1, idx0, idx1,
                  sidx0, sidx1, rows0, rows1, zb_v, acc_sh,
                  msem0, msem1, gsem0, gsem1, ssem0, ssem1):
    cid = lax.axis_index("c")
    sid = lax.axis_index("s")
    srcb = (src0, src1)
    dstb = (dst0, dst1)
    ewb = (ew0, ew1)
    idxb = (idx0, idx1)
    sidxb = (sidx0, sidx1)
    rowsb = (rows0, rows1)
    msem = (msem0, msem1)
    gsem = (gsem0, gsem1)
    ssem = (ssem0, ssem1)

    @pl.loop(0, ZB)
    def _(i):
        zb_v[i, :] = jnp.zeros((16,), jnp.float32)

    epw = E_PAD // NS
    n2 = epw // EDGE_K // 2
    tile0 = pl.multiple_of(sid * epw, 8)

    def load_meta(b, k):
        base = pl.multiple_of(tile0 + k * EDGE_K, 8)
        pltpu.async_copy(src_hbm.at[pl.ds(base, EDGE_K)], srcb[b], msem[b])
        pltpu.async_copy(dst_hbm.at[pl.ds(base, EDGE_K)], dstb[b], msem[b])
        pltpu.async_copy(ew_hbm.at[pl.ds(base, EDGE_K)], ewb[b], msem[b])

    def wait_meta(b):
        pltpu.make_async_copy(src_hbm.at[pl.ds(0, EDGE_K)], srcb[b],
                              msem[b]).wait()
        pltpu.make_async_copy(dst_hbm.at[pl.ds(0, EDGE_K)], dstb[b],
                              msem[b]).wait()
        pltpu.make_async_copy(ew_hbm.at[pl.ds(0, EDGE_K)], ewb[b],
                              msem[b]).wait()

    def fire_gather(b):
        pltpu.async_copy(y_hbm.at[idxb[b]], rowsb[b], gsem[b])

    def wait_gather(b):
        pltpu.make_async_copy(y_hbm.at[idxb[b]], rowsb[b], gsem[b]).wait()

    def fire_scatter(b):
        pltpu.async_copy(rowsb[b], acc_sh.at[sidxb[b]], ssem[b], add=True)

    def wait_scatter(b):
        pltpu.make_async_copy(rowsb[b], acc_sh.at[sidxb[b]], ssem[b]).wait()

    def compute_idx(b, c):
        @plsc.parallel_loop(0, EDGE_K // 16, unroll=4)
        def _(i):
            o = pl.multiple_of(i * 16, 8)
            idxb[b][pl.ds(o, 16)] = srcb[b][pl.ds(o, 16)] * 4 + c

    def scale_and_scatter(b):
        @plsc.parallel_loop(0, EDGE_K // 16, unroll=2)
        def _(i):
            o = pl.multiple_of(i * 16, 8)
            w = ewb[b][pl.ds(o, 16)]
            sidxb[b][pl.ds(o, 16)] = dstb[b][pl.ds(o, 16)]
            for j in range(16):
                rowsb[b][o + j, :] = rowsb[b][o + j, :] * w[j]
        fire_scatter(b)

    for kp in range(2):
        c = cid * 2 + kp

        @pl.loop(0, ROWS_PER_TILE // ZB)
        def _(j):
            r0 = pl.multiple_of(sid * ROWS_PER_TILE + j * ZB, 8)
            pltpu.async_copy(zb_v, acc_sh.at[pl.ds(r0, ZB)], msem0)

        @pl.loop(0, ROWS_PER_TILE // ZB)
        def _(j):
            pltpu.make_async_copy(zb_v, acc_sh.at[pl.ds(0, ZB)],
                                  msem0).wait()

        plsc.subcore_barrier()

        load_meta(0, 0)
        load_meta(1, 1)
        wait_meta(0)
        compute_idx(0, c)
        fire_gather(0)

        @pl.loop(0, n2)
        def _(j):
            wait_meta(1)
            compute_idx(1, c)

            @pl.when(j > 0)
            def _():
                wait_scatter(1)

            fire_gather(1)
            wait_gather(0)
            scale_and_scatter(0)

            @pl.when(j < n2 - 1)
            def _():
                load_meta(0, 2 * j + 2)
                wait_meta(0)
                compute_idx(0, c)
                wait_scatter(0)
                fire_gather(0)

            wait_gather(1)
            scale_and_scatter(1)

            @pl.when(j < n2 - 1)
            def _():
                load_meta(1, 2 * j + 3)

        wait_scatter(0)
        wait_scatter(1)
        plsc.subcore_barrier()

        @pl.loop(0, ROWS_PER_TILE // ZB)
        def _(j):
            r0 = pl.multiple_of(sid * ROWS_PER_TILE + j * ZB, 8)
            pltpu.async_copy(acc_sh.at[pl.ds(r0, ZB)],
                             out_hbm.at[pl.ds(r0, ZB), pl.ds(c * 16, 16)],
                             msem0)

        @pl.loop(0, ROWS_PER_TILE // ZB)
        def _(j):
            pltpu.make_async_copy(acc_sh.at[pl.ds(0, ZB)],
                                  out_hbm.at[pl.ds(0, ZB), pl.ds(0, 16)],
                                  msem0).wait()


def _sc_conv(y_flat, src, dst, ew):
    mesh = plsc.VectorSubcoreMesh(core_axis_name="c", subcore_axis_name="s")
    return pl.kernel(
        _sc_conv_body,
        out_type=jax.ShapeDtypeStruct((N_PAD, H), jnp.float32),
        mesh=mesh,
        compiler_params=pltpu.CompilerParams(use_tc_tiling_on_sc=False),
        scratch_types=[
            pltpu.VMEM((EDGE_K,), jnp.int32),
            pltpu.VMEM((EDGE_K,), jnp.int32),
            pltpu.VMEM((EDGE_K,), jnp.int32),
            pltpu.VMEM((EDGE_K,), jnp.int32),
            pltpu.VMEM((EDGE_K,), jnp.float32),
            pltpu.VMEM((EDGE_K,), jnp.float32),
            pltpu.VMEM((EDGE_K,), jnp.int32),
            pltpu.VMEM((EDGE_K,), jnp.int32),
            pltpu.VMEM((EDGE_K,), jnp.int32),
            pltpu.VMEM((EDGE_K,), jnp.int32),
            pltpu.VMEM((EDGE_K, 16), jnp.float32),
            pltpu.VMEM((EDGE_K, 16), jnp.float32),
            pltpu.VMEM((ZB, 16), jnp.float32),
            pltpu.VMEM_SHARED((N_PAD, 16), jnp.float32),
            pltpu.SemaphoreType.DMA,
            pltpu.SemaphoreType.DMA,
            pltpu.SemaphoreType.DMA,
            pltpu.SemaphoreType.DMA,
            pltpu.SemaphoreType.DMA,
            pltpu.SemaphoreType.DMA,
        ],
    )(y_flat, src, dst, ew)


def kernel(agent_hist, lane_nodes, edge_index_aa, edges_length, edge_index_al,
           edges_length_al, Wih_a, Whh_a, bih_a, bhh_a, Wih_l, Whh_l, bih_l,
           bhh_l, we, be, W1, b1, W2, b2, bn_gamma, bn_beta, bn_mean, bn_var):
    agent_emb = _run_lstm(agent_hist, Wih_a, Whh_a, bih_a, bhh_a, False, LSTM_BLOCK)
    lane_emb = _run_lstm(lane_nodes, Wih_l, Whh_l, bih_l, bhh_l, True, LSTM_BLOCK)
    x = jnp.concatenate([agent_emb, lane_emb], axis=0)
    x_pad = jnp.pad(x, ((0, N_PAD - N_NODES), (0, 0)))

    lens = jnp.concatenate([edges_length, edges_length_al])
    w_all = _edge_weights(lens, we, be)
    w_aa = w_all[:E_AA]
    w_al = w_all[E_AA:]

    al_a = edge_index_al[0].astype(jnp.int32)
    al_l = edge_index_al[1].astype(jnp.int32) + A_N
    n_pad_e = E_PAD - E_REAL
    pad_idx = jnp.arange(n_pad_e, dtype=jnp.int32) % N_NODES
    src = jnp.concatenate([edge_index_aa[0].astype(jnp.int32), al_a, al_l,
                           pad_idx])
    dst = jnp.concatenate([edge_index_aa[1].astype(jnp.int32), al_l, al_a,
                           pad_idx])
    ew = jnp.concatenate([w_aa, w_al, w_al,
                          jnp.zeros((n_pad_e,), jnp.float32)])

    degp = _sc_degree(dst, ew)
    d0 = degp[0].reshape(N_PAD, 1)
    d1 = degp[1].reshape(N_PAD, 1)

    y1, xw1, dinv = _glue1(x_pad, W1.T, d0, d1)
    s1 = _sc_conv(y1.reshape(4 * N_PAD, 16), src, dst, ew)

    bn_scale = bn_gamma / jnp.sqrt(bn_var + 1e-5)
    bn_shift = bn_beta - bn_mean * bn_scale
    y2, xw2 = _glue2(s1, xw1, dinv, W2.T, b1.reshape(1, H),
                     bn_scale.reshape(1, H), bn_shift.reshape(1, H))
    s2 = _sc_conv(y2.reshape(4 * N_PAD, 16), src, dst, ew)
    x_out = _glue3(s2, xw2, dinv, b2.reshape(1, H))

    agent_map = x_out[:A_N]
    lane_out = x_out[A_N:N_NODES]
    return (agent_emb, agent_map, lane_emb, lane_out)
